# baseline (speedup 1.0000x reference)
import functools

import jax
import jax.numpy as jnp
from jax import lax
from jax.experimental import pallas as pl
from jax.experimental.pallas import tpu as pltpu
from jax.experimental.pallas import tpu_sc as plsc


def _sc_gather_pairs(table2, idx2):
    info = plsc.get_sparse_core_info()
    nc, ns = info.num_cores, info.num_subcores
    nw = nc * ns
    b, d2 = idx2.shape[0], table2.shape[1]
    b_per_w = b // nw
    mesh = plsc.VectorSubcoreMesh(core_axis_name="c", subcore_axis_name="s")

    @functools.partial(
        pl.kernel,
        mesh=mesh,
        out_type=jax.ShapeDtypeStruct((b, d2), jnp.float32),
        scratch_types=[
            pltpu.VMEM((b_per_w,), jnp.int32),
            pltpu.VMEM((b_per_w, d2), jnp.float32),
            pltpu.SemaphoreType.DMA,
        ],
    )
    def gather_kernel(table_hbm, idx_hbm, out_hbm, idx_v, rows_v, sem):
        wid = lax.axis_index("s") * nc + lax.axis_index("c")
        base = wid * b_per_w
        pltpu.sync_copy(idx_hbm.at[pl.ds(base, b_per_w)], idx_v)
        pltpu.async_copy(table_hbm.at[idx_v], rows_v, sem).wait()
        pltpu.sync_copy(rows_v, out_hbm.at[pl.ds(base, b_per_w)])

    return gather_kernel(table2, idx2)


def _mm_body(pairs_ref, par_ref, w_ref, b_ref, o_ref, e_scr):
    d = e_scr.shape[1]

    @pl.when(pl.program_id(0) == 0)
    def _():
        e_scr[...] = jnp.where(
            par_ref[...] == 1, pairs_ref[:, d:], pairs_ref[:, :d]
        )

    o_ref[...] = (
        lax.dot_general(
            e_scr[...],
            w_ref[...],
            (((1,), (1,)), ((), ())),
            preferred_element_type=jnp.float32,
        )
        + b_ref[...]
    )


def _tc_project(pairs, parity, W, b, vblk=2048):
    bsz, d2 = pairs.shape
    d = d2 // 2
    vocab = W.shape[0]
    nv = pl.cdiv(vocab, vblk)
    return pl.pallas_call(
        _mm_body,
        grid=(nv,),
        in_specs=[
            pl.BlockSpec((bsz, d2), lambda i: (0, 0)),
            pl.BlockSpec((bsz, 1), lambda i: (0, 0)),
            pl.BlockSpec((vblk, d), lambda i: (i, 0)),
            pl.BlockSpec((1, vblk), lambda i: (0, i)),
        ],
        out_specs=pl.BlockSpec((bsz, vblk), lambda i: (0, i)),
        out_shape=jax.ShapeDtypeStruct((bsz, vocab), jnp.float32),
        scratch_shapes=[pltpu.VMEM((bsz, d), jnp.float32)],
    )(pairs, parity, W, b.reshape(1, vocab))


def kernel(inputs, table, W, b):
    vocab, d = table.shape
    table2 = table.reshape(vocab // 2, 2 * d)
    pairs = _sc_gather_pairs(table2, inputs >> 1)
    parity = (inputs & 1).reshape(inputs.shape[0], 1)
    return _tc_project(pairs, parity, W, b)
